# baseline (speedup 1.0000x reference)
import functools

import jax
import jax.numpy as jnp
from jax import lax
from jax.experimental import pallas as pl
from jax.experimental.pallas import tpu as pltpu
from jax.experimental.pallas import tpu_sc as plsc

_B = 64
_N = 512
_TOT = _B * _N
_NW = 32
_PER_W = _TOT // _NW
_LANES = 16
_TAB = 128
_NCHUNK = _TAB // _LANES
_BB = 8
_SPLIT = 48
_NB_SC = _B - _SPLIT
_LOG2E = 1.4426950408889634


def _sc_gather_body(nums_hbm, a_tab_hbm, na_tab_hbm, z_tab_hbm,
                    a_out_hbm, na_out_hbm, z_out_hbm,
                    idx_v, a_v, na_v, z_v, a_tab_v, na_tab_v, z_tab_v):
    wid = lax.axis_index("s") * 2 + lax.axis_index("c")
    base = wid * _PER_W
    pltpu.sync_copy(a_tab_hbm, a_tab_v)
    pltpu.sync_copy(na_tab_hbm, na_tab_v)
    pltpu.sync_copy(z_tab_hbm, z_tab_v)
    pltpu.sync_copy(nums_hbm.at[pl.ds(base, _PER_W)], idx_v)

    def body(i, carry):
        off = i * jnp.int32(_LANES)
        idx = idx_v[pl.ds(off, _LANES)]
        lo = lax.bitwise_and(idx, jnp.int32(_LANES - 1))
        hi = lax.shift_right_logical(idx, jnp.int32(4))
        acc_a = jnp.zeros((_LANES,), jnp.float32)
        acc_na = jnp.zeros((_LANES,), jnp.float32)
        acc_z = jnp.zeros((_LANES,), jnp.float32)
        for k in range(_NCHUNK):
            ch_a = a_tab_v[pl.ds(k * _LANES, _LANES)]
            ch_na = na_tab_v[pl.ds(k * _LANES, _LANES)]
            ch_z = z_tab_v[pl.ds(k * _LANES, _LANES)]
            ga = ch_a.at[lo].get(mode="promise_in_bounds")
            gna = ch_na.at[lo].get(mode="promise_in_bounds")
            gz = ch_z.at[lo].get(mode="promise_in_bounds")
            m = hi == jnp.int32(k)
            acc_a = jnp.where(m, ga, acc_a)
            acc_na = jnp.where(m, gna, acc_na)
            acc_z = jnp.where(m, gz, acc_z)
        a_v[pl.ds(off, _LANES)] = acc_a
        na_v[pl.ds(off, _LANES)] = acc_na
        z_v[pl.ds(off, _LANES)] = acc_z
        return carry

    lax.fori_loop(jnp.int32(0), jnp.int32(_PER_W // _LANES), body,
                  jnp.int32(0))
    pltpu.sync_copy(a_v, a_out_hbm.at[pl.ds(base, _PER_W)])
    pltpu.sync_copy(na_v, na_out_hbm.at[pl.ds(base, _PER_W)])
    pltpu.sync_copy(z_v, z_out_hbm.at[pl.ds(base, _PER_W)])


@functools.lru_cache(maxsize=1)
def _sc_gather():
    return pl.kernel(
        _sc_gather_body,
        out_type=[jax.ShapeDtypeStruct((_TOT,), jnp.float32)] * 3,
        mesh=plsc.VectorSubcoreMesh(core_axis_name="c", subcore_axis_name="s"),
        scratch_types=[
            pltpu.VMEM((_PER_W,), jnp.int32),
            pltpu.VMEM((_PER_W,), jnp.float32),
            pltpu.VMEM((_PER_W,), jnp.float32),
            pltpu.VMEM((_PER_W,), jnp.float32),
            pltpu.VMEM((_TAB,), jnp.float32),
            pltpu.VMEM((_TAB,), jnp.float32),
            pltpu.VMEM((_TAB,), jnp.float32),
        ],
    )


def _newton_rsqrt(v):
    i = lax.bitcast_convert_type(v, jnp.int32)
    i = jnp.int32(0x5F3759DF) - lax.shift_right_logical(i, jnp.int32(1))
    y = lax.bitcast_convert_type(i, jnp.float32)
    hv = v * jnp.float32(0.5)
    for _ in range(2):
        y = y * (jnp.float32(1.5) - hv * y * y)
    return y


def _sc_pair_body(d_hbm, a_hbm, z_hbm, part_hbm,
                  a_all_v, z_all_v, slab_v, partials_v):
    wid = lax.axis_index("s") * 2 + lax.axis_index("c")
    wrow = wid * jnp.int32(_LANES)

    def batch_body(b, carry):
        gb = jnp.int32(_SPLIT) + b
        base = gb * jnp.int32(_N)
        pltpu.sync_copy(a_hbm.at[pl.ds(base, _N)], a_all_v)
        pltpu.sync_copy(z_hbm.at[pl.ds(base, _N)], z_all_v)
        pltpu.sync_copy(d_hbm.at[gb, pl.ds(wrow, _LANES), :], slab_v)
        amine = a_all_v[pl.ds(wrow, _LANES)]
        zmine = z_all_v[pl.ds(wrow, _LANES)]
        acc_b = jnp.zeros((_LANES,), jnp.float32)
        for r in range(_LANES):
            sel = jnp.full((_LANES,), r, jnp.int32)
            na_r = jnp.float32(0.0) - amine.at[sel].get(
                mode="promise_in_bounds")
            z_r = zmine.at[sel].get(mode="promise_in_bounds")

            def col_body(j, accs, r=r, na_r=na_r, z_r=z_r):
                base4 = j * jnp.int32(4 * _LANES)
                new = []
                for k in range(4):
                    off = base4 + jnp.int32(k * _LANES)
                    v = slab_v[r, pl.ds(off, _LANES)]
                    aj = a_all_v[pl.ds(off, _LANES)]
                    zj = z_all_v[pl.ds(off, _LANES)]
                    y = _newton_rsqrt(v)
                    d15 = v * (v * y)
                    e = jnp.exp((na_r * aj) * d15)
                    new.append(accs[k] + e * (z_r * zj) * (y * y))
                return tuple(new)

            accs = lax.fori_loop(jnp.int32(0), jnp.int32(_N // (4 * _LANES)),
                                 col_body,
                                 tuple(jnp.zeros((_LANES,), jnp.float32)
                                       for _ in range(4)))
            acc_b = acc_b + (accs[0] + accs[1]) + (accs[2] + accs[3])
        partials_v[pl.ds(b * jnp.int32(_LANES), _LANES)] = acc_b
        return carry

    lax.fori_loop(jnp.int32(0), jnp.int32(_NB_SC), batch_body, jnp.int32(0))
    pltpu.sync_copy(partials_v, part_hbm.at[wid])


@functools.lru_cache(maxsize=1)
def _sc_pair():
    return pl.kernel(
        _sc_pair_body,
        out_type=jax.ShapeDtypeStruct((_NW, _NB_SC * _LANES), jnp.float32),
        mesh=plsc.VectorSubcoreMesh(core_axis_name="c", subcore_axis_name="s"),
        scratch_types=[
            pltpu.VMEM((_N,), jnp.float32),
            pltpu.VMEM((_N,), jnp.float32),
            pltpu.VMEM((_LANES, _N), jnp.float32),
            pltpu.VMEM((_NB_SC * _LANES,), jnp.float32),
        ],
    )


def _tc_body(na_ref, a_ref, z_ref, d_ref, o_ref):
    for t in range(_BB):
        na = na_ref[t]
        a = a_ref[t]
        z = z_ref[t]
        d = d_ref[t]
        nac = jnp.reshape(na, (_N, 1))
        zc = jnp.reshape(z, (_N, 1))
        alpha2 = nac * a
        zz = zc * z
        r = lax.rsqrt(d)
        p = d * r
        d15 = d * p
        e = jnp.exp2(alpha2 * d15) * zz * (r * r)
        o_ref[t] = jnp.sum(e, axis=(0, 1), keepdims=True)


def _bzz(b):
    z = jnp.int32(0)
    return (b, z, z)


def _tc_reduce_body(p_ref, o_ref):
    s = jnp.sum(p_ref[...], axis=(0, 2))
    o_ref[...] = jnp.reshape(s, (1, 1, _NB_SC))


def kernel(numbers, d_ij, weight):
    nums = numbers.reshape(-1).astype(jnp.int32)
    w = weight.astype(jnp.float32)
    pad = _TAB - w.shape[0]
    a_tab = jnp.pad(w[:, 0], (0, pad))
    na_tab = a_tab * jnp.float32(-_LOG2E)
    z_tab = jnp.pad(w[:, 1], (0, pad))

    a_g, na_g, z_g = _sc_gather()(nums, a_tab, na_tab, z_tab)

    part = _sc_pair()(d_ij, a_g, z_g)

    out_tc = pl.pallas_call(
        _tc_body,
        grid=(_SPLIT // _BB,),
        in_specs=[
            pl.BlockSpec((_BB, 1, _N), _bzz),
            pl.BlockSpec((_BB, 1, _N), _bzz),
            pl.BlockSpec((_BB, 1, _N), _bzz),
            pl.BlockSpec((_BB, _N, _N), _bzz),
        ],
        out_specs=pl.BlockSpec((_BB, 1, 1), _bzz),
        out_shape=jax.ShapeDtypeStruct((_SPLIT, 1, 1), jnp.float32),
        compiler_params=pltpu.CompilerParams(
            dimension_semantics=("arbitrary",),
        ),
    )(na_g.reshape(_B, 1, _N), a_g.reshape(_B, 1, _N),
      z_g.reshape(_B, 1, _N), d_ij)

    out_sc = pl.pallas_call(
        _tc_reduce_body,
        out_shape=jax.ShapeDtypeStruct((1, 1, _NB_SC), jnp.float32),
    )(part.reshape(_NW, _NB_SC, _LANES))

    out = jnp.concatenate([out_tc.reshape(_SPLIT), out_sc.reshape(_NB_SC)])
    return out.astype(jnp.float64)


# speedup vs baseline: 2.1649x; 2.1649x over previous
import functools

import jax
import jax.numpy as jnp
from jax import lax
from jax.experimental import pallas as pl
from jax.experimental.pallas import tpu as pltpu
from jax.experimental.pallas import tpu_sc as plsc

_B = 64
_N = 512
_TOT = _B * _N
_NW = 32
_PER_W = _TOT // _NW
_LANES = 16
_TAB = 128
_NCHUNK = _TAB // _LANES
_BB = 8
_LOG2E = 1.4426950408889634


def _sc_gather_body(nums_hbm, a_tab_hbm, na_tab_hbm, z_tab_hbm,
                    a_out_hbm, na_out_hbm, z_out_hbm,
                    idx_v, a_v, na_v, z_v, a_tab_v, na_tab_v, z_tab_v):
    wid = lax.axis_index("s") * 2 + lax.axis_index("c")
    base = wid * _PER_W
    pltpu.sync_copy(a_tab_hbm, a_tab_v)
    pltpu.sync_copy(na_tab_hbm, na_tab_v)
    pltpu.sync_copy(z_tab_hbm, z_tab_v)
    pltpu.sync_copy(nums_hbm.at[pl.ds(base, _PER_W)], idx_v)

    def body(i, carry):
        off = i * jnp.int32(_LANES)
        idx = idx_v[pl.ds(off, _LANES)]
        lo = lax.bitwise_and(idx, jnp.int32(_LANES - 1))
        hi = lax.shift_right_logical(idx, jnp.int32(4))
        acc_a = jnp.zeros((_LANES,), jnp.float32)
        acc_na = jnp.zeros((_LANES,), jnp.float32)
        acc_z = jnp.zeros((_LANES,), jnp.float32)
        for k in range(_NCHUNK):
            ch_a = a_tab_v[pl.ds(k * _LANES, _LANES)]
            ch_na = na_tab_v[pl.ds(k * _LANES, _LANES)]
            ch_z = z_tab_v[pl.ds(k * _LANES, _LANES)]
            ga = ch_a.at[lo].get(mode="promise_in_bounds")
            gna = ch_na.at[lo].get(mode="promise_in_bounds")
            gz = ch_z.at[lo].get(mode="promise_in_bounds")
            m = hi == jnp.int32(k)
            acc_a = jnp.where(m, ga, acc_a)
            acc_na = jnp.where(m, gna, acc_na)
            acc_z = jnp.where(m, gz, acc_z)
        a_v[pl.ds(off, _LANES)] = acc_a
        na_v[pl.ds(off, _LANES)] = acc_na
        z_v[pl.ds(off, _LANES)] = acc_z
        return carry

    lax.fori_loop(jnp.int32(0), jnp.int32(_PER_W // _LANES), body,
                  jnp.int32(0))
    pltpu.sync_copy(a_v, a_out_hbm.at[pl.ds(base, _PER_W)])
    pltpu.sync_copy(na_v, na_out_hbm.at[pl.ds(base, _PER_W)])
    pltpu.sync_copy(z_v, z_out_hbm.at[pl.ds(base, _PER_W)])


@functools.lru_cache(maxsize=1)
def _sc_gather():
    return pl.kernel(
        _sc_gather_body,
        out_type=[jax.ShapeDtypeStruct((_TOT,), jnp.float32)] * 3,
        mesh=plsc.VectorSubcoreMesh(core_axis_name="c", subcore_axis_name="s"),
        scratch_types=[
            pltpu.VMEM((_PER_W,), jnp.int32),
            pltpu.VMEM((_PER_W,), jnp.float32),
            pltpu.VMEM((_PER_W,), jnp.float32),
            pltpu.VMEM((_PER_W,), jnp.float32),
            pltpu.VMEM((_TAB,), jnp.float32),
            pltpu.VMEM((_TAB,), jnp.float32),
            pltpu.VMEM((_TAB,), jnp.float32),
        ],
    )


def _tc_body(na_ref, a_ref, z_ref, d_ref, o_ref):
    for t in range(_BB):
        na = na_ref[t]
        a = a_ref[t]
        z = z_ref[t]
        d = d_ref[t]
        nac = jnp.reshape(na, (_N, 1))
        zc = jnp.reshape(z, (_N, 1))
        alpha2 = nac * a
        zz = zc * z
        r = lax.rsqrt(d)
        p = d * r
        d15 = d * p
        e = jnp.exp2(alpha2 * d15) * zz * (r * r)
        o_ref[t] = jnp.sum(e, axis=(0, 1), keepdims=True)


def _bzz(b):
    z = jnp.int32(0)
    return (b, z, z)


def kernel(numbers, d_ij, weight):
    nums = numbers.reshape(-1).astype(jnp.int32)
    w = weight.astype(jnp.float32)
    pad = _TAB - w.shape[0]
    a_tab = jnp.pad(w[:, 0], (0, pad))
    na_tab = a_tab * jnp.float32(-_LOG2E)
    z_tab = jnp.pad(w[:, 1], (0, pad))

    a_g, na_g, z_g = _sc_gather()(nums, a_tab, na_tab, z_tab)

    out = pl.pallas_call(
        _tc_body,
        grid=(_B // _BB,),
        in_specs=[
            pl.BlockSpec((_BB, 1, _N), _bzz),
            pl.BlockSpec((_BB, 1, _N), _bzz),
            pl.BlockSpec((_BB, 1, _N), _bzz),
            pl.BlockSpec((_BB, _N, _N), _bzz),
        ],
        out_specs=pl.BlockSpec((_BB, 1, 1), _bzz),
        out_shape=jax.ShapeDtypeStruct((_B, 1, 1), jnp.float32),
        compiler_params=pltpu.CompilerParams(
            dimension_semantics=("arbitrary",),
        ),
    )(na_g.reshape(_B, 1, _N), a_g.reshape(_B, 1, _N),
      z_g.reshape(_B, 1, _N), d_ij)

    return out.reshape(_B).astype(jnp.float64)
